# baseline (speedup 1.0000x reference)
import functools

import jax
import jax.numpy as jnp
from jax import lax
from jax.experimental import pallas as pl
from jax.experimental.pallas import tpu as pltpu
from jax.experimental.pallas import tpu_sc as plsc

N = 100000
E = 3200000
NPAD = 100096
NB = 2000
NW = 32
EPW = E // NW
WIN = 400
NWIN = EPW // WIN
RPT = NPAD // 16

_mesh = lambda: plsc.VectorSubcoreMesh(core_axis_name="c", subcore_axis_name="s")


@functools.cache
def _make_dense(F_in, H, D, chunked_in):
    F = H * D
    nch = F // 16

    def body(*refs):
        n_in = max(F_in // 16, 1) if chunked_in else 1
        xrefs = refs[:n_in]
        W_ref, As_ref, Ad_ref = refs[n_in:n_in + 3]
        outs = refs[n_in + 3:]
        chunk_refs = outs[:nch]
        as_ref = outs[nch]
        ad_ref = outs[nch + 1]
        wself_ref = outs[nch + 2]
        if chunked_in:
            x = jnp.concatenate([r[...] for r in xrefs], axis=1)
        else:
            x = xrefs[0][...]
        h = jnp.dot(x, W_ref[...], preferred_element_type=jnp.float32)
        asv = jnp.dot(h, As_ref[...], preferred_element_type=jnp.float32)
        adv = jnp.dot(h, Ad_ref[...], preferred_element_type=jnp.float32)
        z = asv + adv
        wself_ref[...] = jnp.exp(jnp.maximum(z, 0.2 * z))
        as_ref[...] = asv
        ad_ref[...] = adv
        for k in range(nch):
            chunk_refs[k][...] = h[:, k * 16:(k + 1) * 16]

    grid = (N // NB,)
    if chunked_in:
        x_specs = [pl.BlockSpec((NB, 16), lambda i: (i, 0))] * (F_in // 16)
    else:
        x_specs = [pl.BlockSpec((NB, F_in), lambda i: (i, 0))]
    in_specs = x_specs + [
        pl.BlockSpec((F_in, F), lambda i: (0, 0)),
        pl.BlockSpec((F, H), lambda i: (0, 0)),
        pl.BlockSpec((F, H), lambda i: (0, 0)),
    ]
    out_specs = ([pl.BlockSpec((NB, 16), lambda i: (i, 0))] * nch
                 + [pl.BlockSpec((NB, H), lambda i: (i, 0))] * 3)
    out_shape = ([jax.ShapeDtypeStruct((N, 16), jnp.float32)] * nch
                 + [jax.ShapeDtypeStruct((N, H), jnp.float32)] * 3)
    return pl.pallas_call(body, grid=grid, in_specs=in_specs,
                          out_specs=out_specs, out_shape=out_shape)


@functools.cache
def _make_attn(H):
    per_set = 1 + 3 * H
    scratch = []
    for _ in range(3):
        scratch += [pltpu.VMEM((2, WIN), jnp.int32)]
        scratch += [pltpu.VMEM((WIN,), jnp.float32) for _ in range(3 * H)]
    scratch += [pltpu.VMEM_SHARED((NPAD,), jnp.float32) for _ in range(H)]
    scratch += [pltpu.VMEM((RPT,), jnp.float32)]
    scratch += [pltpu.SemaphoreType.DMA] * 9

    @functools.partial(
        pl.kernel, mesh=_mesh(),
        out_type=[jax.ShapeDtypeStruct((H * E,), jnp.float32),
                  jax.ShapeDtypeStruct((2 * H * NPAD,), jnp.float32)],
        scratch_types=scratch,
        compiler_params=pltpu.CompilerParams(use_tc_tiling_on_sc=False))
    def attn(edges_hbm, *rest):
        atabs = rest[:2 * H]
        zn_hbm = rest[2 * H]
        w_out, den_out = rest[2 * H + 1], rest[2 * H + 2]
        sc = rest[2 * H + 3:]
        sets = []
        for b in range(3):
            o = b * per_set
            sets.append(dict(ebuf=sc[o],
                             asb=sc[o + 1:o + 1 + H],
                             adb=sc[o + 1 + H:o + 1 + 2 * H],
                             wb=sc[o + 1 + 2 * H:o + 1 + 3 * H]))
        dens = sc[3 * per_set:3 * per_set + H]
        vbuf = sc[3 * per_set + H]
        gsem = sc[3 * per_set + H + 1:3 * per_set + H + 4]
        ssem = sc[3 * per_set + H + 4:3 * per_set + H + 7]
        wsem = sc[3 * per_set + H + 7:3 * per_set + H + 10]
        cid = lax.axis_index("c")
        sid = lax.axis_index("s")
        wid = sid * 2 + cid
        pltpu.sync_copy(zn_hbm.at[pl.ds(sid * RPT, RPT)], vbuf)
        for hh in range(H):
            pltpu.sync_copy(vbuf, dens[hh].at[pl.ds(sid * RPT, RPT)])
        plsc.subcore_barrier()

        def issue(j, b):
            s = sets[b]
            pltpu.sync_copy(edges_hbm.at[wid * NWIN + j], s["ebuf"])
            for hh in range(H):
                pltpu.async_copy(atabs[hh].at[s["ebuf"].at[0]], s["asb"][hh],
                                 gsem[b])
                pltpu.async_copy(atabs[H + hh].at[s["ebuf"].at[1]],
                                 s["adb"][hh], gsem[b])

        def drain(b):
            s = sets[b]
            for hh in range(H):
                pltpu.make_async_copy(atabs[hh].at[s["ebuf"].at[0]],
                                      s["asb"][hh], gsem[b]).wait()
                pltpu.make_async_copy(atabs[H + hh].at[s["ebuf"].at[1]],
                                      s["adb"][hh], gsem[b]).wait()

        def retire(b, jprev):
            s = sets[b]
            basep = wid * EPW + jprev * WIN
            for hh in range(H):
                pltpu.make_async_copy(s["wb"][hh],
                                      dens[hh].at[s["ebuf"].at[1]],
                                      ssem[b]).wait()
                pltpu.make_async_copy(s["wb"][hh],
                                      w_out.at[pl.ds(hh * E + basep, WIN)],
                                      wsem[b]).wait()

        def work(j, b):
            s = sets[b]
            base = wid * EPW + j * WIN
            drain(b)
            for hh in range(H):
                def grp(g, c2, hh=hh):
                    z = (s["asb"][hh][pl.ds(g * 16, 16)]
                         + s["adb"][hh][pl.ds(g * 16, 16)])
                    s["wb"][hh][pl.ds(g * 16, 16)] = jnp.exp(
                        jnp.maximum(z, 0.2 * z))
                    return c2

                lax.fori_loop(0, WIN // 16, grp, 0)
                pltpu.async_copy(s["wb"][hh], dens[hh].at[s["ebuf"].at[1]],
                                 ssem[b], add=True)
                pltpu.async_copy(s["wb"][hh],
                                 w_out.at[pl.ds(hh * E + base, WIN)],
                                 wsem[b])

        issue(0, 0)
        issue(1, 1)
        work(0, 0)
        issue(2, 2)
        work(1, 1)

        def triple(k, carry):
            j = 2 + 3 * k
            for t in range(3):
                b = (2 + t) % 3
                retire((b + 1) % 3, j + t - 2)
                issue(j + t + 1, (b + 1) % 3)
                work(j + t, b)
            return carry

        lax.fori_loop(0, (NWIN - 4) // 3, triple, 0)
        retire(0, NWIN - 4)
        issue(NWIN - 1, 0)
        work(NWIN - 2, 2)
        work(NWIN - 1, 0)
        retire(0, NWIN - 1)
        retire(1, NWIN - 3)
        retire(2, NWIN - 2)
        plsc.subcore_barrier()
        for hh in range(H):
            pltpu.sync_copy(dens[hh].at[pl.ds(sid * RPT, RPT)], vbuf)
            pltpu.sync_copy(
                vbuf,
                den_out.at[pl.ds((cid * H + hh) * NPAD + sid * RPT, RPT)])

    return attn


@functools.cache
def _make_agg(H, hh):
    NQ, TAIL = RPT // WIN, RPT % WIN
    scratch = []
    for _ in range(3):
        scratch += [pltpu.VMEM((2, WIN), jnp.int32),
                    pltpu.VMEM((WIN,), jnp.float32),
                    pltpu.VMEM((WIN, 16), jnp.float32)]
    scratch += [pltpu.VMEM_SHARED((NPAD, 16), jnp.float32)]
    scratch += [pltpu.SemaphoreType.DMA] * 6

    @functools.partial(
        pl.kernel, mesh=_mesh(),
        out_type=jax.ShapeDtypeStruct((2, NPAD, 16), jnp.float32),
        scratch_types=scratch,
        compiler_params=pltpu.CompilerParams(use_tc_tiling_on_sc=False))
    def agg(edges_hbm, w_hbm, table_hbm, z16_hbm, out_hbm, *sc):
        sets = [(sc[3 * b], sc[3 * b + 1], sc[3 * b + 2]) for b in range(3)]
        acc = sc[9]
        gsem = sc[10:13]
        ssem = sc[13:16]
        cid = lax.axis_index("c")
        sid = lax.axis_index("s")
        wid = sid * 2 + cid
        rows0 = sets[0][2]
        pltpu.sync_copy(z16_hbm.at[pl.ds(sid * RPT, WIN)], rows0)
        for q in range(NQ):
            pltpu.sync_copy(rows0, acc.at[pl.ds(sid * RPT + q * WIN, WIN)])
        pltpu.sync_copy(rows0.at[pl.ds(0, TAIL)],
                        acc.at[pl.ds(sid * RPT + NQ * WIN, TAIL)])
        plsc.subcore_barrier()

        def issue(j, b):
            eb, wv, rows = sets[b]
            base = wid * EPW + j * WIN
            pltpu.sync_copy(edges_hbm.at[wid * NWIN + j], eb)
            pltpu.async_copy(w_hbm.at[pl.ds(hh * E + base, WIN)], wv,
                             gsem[b])
            pltpu.async_copy(table_hbm.at[eb.at[0]], rows, gsem[b])

        def retire(b):
            eb, wv, rows = sets[b]
            pltpu.make_async_copy(rows, acc.at[eb.at[1]], ssem[b]).wait()

        def work(j, b):
            eb, wv, rows = sets[b]
            base = wid * EPW + j * WIN
            pltpu.make_async_copy(w_hbm.at[pl.ds(hh * E + base, WIN)], wv,
                                  gsem[b]).wait()
            pltpu.make_async_copy(table_hbm.at[eb.at[0]], rows,
                                  gsem[b]).wait()

            def scale(g, c2):
                wv_arr = wv[pl.ds(g * 16, 16)]
                for ei in range(16):
                    r = g * 16 + ei
                    rows[r, :] = rows[r, :] * wv_arr[ei]
                return c2

            lax.fori_loop(0, WIN // 16, scale, 0)
            pltpu.async_copy(rows, acc.at[eb.at[1]], ssem[b], add=True)

        issue(0, 0)
        issue(1, 1)
        work(0, 0)
        issue(2, 2)
        work(1, 1)

        def triple(k, carry):
            j = 2 + 3 * k
            for t in range(3):
                b = (2 + t) % 3
                retire((b + 1) % 3)
                issue(j + t + 1, (b + 1) % 3)
                work(j + t, b)
            return carry

        lax.fori_loop(0, (NWIN - 4) // 3, triple, 0)
        retire(0)
        issue(NWIN - 1, 0)
        work(NWIN - 2, 2)
        work(NWIN - 1, 0)
        for b in range(3):
            retire(b)
        plsc.subcore_barrier()
        for q in range(NQ):
            pltpu.sync_copy(acc.at[pl.ds(sid * RPT + q * WIN, WIN)], rows0)
            pltpu.sync_copy(rows0,
                            out_hbm.at[cid, pl.ds(sid * RPT + q * WIN, WIN)])
        pltpu.sync_copy(acc.at[pl.ds(sid * RPT + NQ * WIN, TAIL)],
                        rows0.at[pl.ds(0, TAIL)])
        pltpu.sync_copy(rows0.at[pl.ds(0, TAIL)],
                        out_hbm.at[cid, pl.ds(sid * RPT + NQ * WIN, TAIL)])

    return agg


@functools.cache
def _make_final(H, D, k):
    F = H * D
    hk = (k * 16) // D

    def body(part_ref, den_ref, wself_ref, chunk_ref, b_ref, out_ref):
        ws = wself_ref[...]
        p = part_ref[0] + part_ref[1]
        num = p + ws[:, hk:hk + 1] * chunk_ref[...]
        dd = den_ref[0, :, hk] + den_ref[1, :, hk] + ws[:, hk] + 1e-16
        o = num / dd[:, None] + b_ref[0, k * 16:(k + 1) * 16]
        out_ref[...] = jnp.where(o > 0, o, jnp.exp(o) - 1.0)

    grid = (N // NB,)
    in_specs = [pl.BlockSpec((2, NB, 16), lambda i: (0, i, 0)),
                pl.BlockSpec((2, NB, H), lambda i: (0, i, 0)),
                pl.BlockSpec((NB, H), lambda i: (i, 0)),
                pl.BlockSpec((NB, 16), lambda i: (i, 0)),
                pl.BlockSpec((1, F), lambda i: (0, 0))]
    return pl.pallas_call(
        body, grid=grid, in_specs=in_specs,
        out_specs=pl.BlockSpec((NB, 16), lambda i: (i, 0)),
        out_shape=jax.ShapeDtypeStruct((N, 16), jnp.float32))


@functools.cache
def _make_mlp():
    def body(xa_ref, xb_ref, W1_ref, b1_ref, W2_ref, b2_ref, W3_ref, b3_ref,
             out_ref):
        x = jnp.concatenate([xa_ref[...], xb_ref[...]], axis=1)
        h = jnp.maximum(
            jnp.dot(x, W1_ref[...],
                    preferred_element_type=jnp.float32) + b1_ref[...], 0.0)
        h = jnp.maximum(
            jnp.dot(h, W2_ref[...],
                    preferred_element_type=jnp.float32) + b2_ref[...], 0.0)
        o = jnp.dot(h, W3_ref[...],
                    preferred_element_type=jnp.float32) + b3_ref[...]
        out_ref[...] = 1.0 / (1.0 + jnp.exp(-o))

    grid = (N // NB,)
    in_specs = [
        pl.BlockSpec((NB, 16), lambda i: (i, 0)),
        pl.BlockSpec((NB, 16), lambda i: (i, 0)),
        pl.BlockSpec((32, 128), lambda i: (0, 0)),
        pl.BlockSpec((1, 128), lambda i: (0, 0)),
        pl.BlockSpec((128, 128), lambda i: (0, 0)),
        pl.BlockSpec((1, 128), lambda i: (0, 0)),
        pl.BlockSpec((128, 1), lambda i: (0, 0)),
        pl.BlockSpec((1, 1), lambda i: (0, 0)),
    ]
    return pl.pallas_call(
        body, grid=grid, in_specs=in_specs,
        out_specs=pl.BlockSpec((NB, 1), lambda i: (i, 0)),
        out_shape=jax.ShapeDtypeStruct((N, 1), jnp.float32))


def _head_expand(a):
    H, D = a.shape
    eye = jnp.eye(H, dtype=a.dtype)
    return (a[:, :, None] * eye[:, None, :]).reshape(H * D, H)


def _gat_layer(hs, edges3, zeros_n, zeros_n16, W, a_s, a_d, b, H, D):
    F = H * D
    chunked_in = isinstance(hs, (list, tuple))
    F_in = sum(hh.shape[1] for hh in hs) if chunked_in else hs.shape[1]
    xs = list(hs) if chunked_in else [hs]
    outs = _make_dense(F_in, H, D, chunked_in)(
        *xs, W, _head_expand(a_s), _head_expand(a_d))
    chunks = outs[:F // 16]
    asv = outs[F // 16]
    adv = outs[F // 16 + 1]
    wself = outs[F // 16 + 2]
    acols = ([asv[:, j] for j in range(H)] + [adv[:, j] for j in range(H)])
    w_all, den = _make_attn(H)(edges3, *acols, zeros_n)
    den = jnp.transpose(den.reshape(2, H, NPAD), (0, 2, 1))
    parts = [_make_agg(H, (k * 16) // D)(edges3, w_all, chunks[k], zeros_n16)
             for k in range(F // 16)]
    bb = b.reshape(1, F)
    return [_make_final(H, D, k)(parts[k], den, wself, chunks[k], bb)
            for k in range(F // 16)]


def kernel(x, edge_index, W1, a1s, a1d, b1, W2, a2s, a2d, b2, W3, a3s, a3d,
           b3, W4, a4s, a4d, b4, Wl1, bl1, Wl2, bl2, Wl3, bl3):
    er = edge_index.reshape(2, NW, NWIN, WIN)
    edges3 = jnp.transpose(er, (1, 2, 0, 3)).reshape(NW * NWIN, 2, WIN)
    zeros_n = jnp.zeros((NPAD,), jnp.float32)
    zeros_n16 = jnp.zeros((NPAD, 16), jnp.float32)
    h = _gat_layer(x, edges3, zeros_n, zeros_n16, W1, a1s, a1d, b1, 2, 32)
    h = _gat_layer(h, edges3, zeros_n, zeros_n16, W2, a2s, a2d, b2, 2, 64)
    h = _gat_layer(h, edges3, zeros_n, zeros_n16, W3, a3s, a3d, b3, 2, 64)
    h = _gat_layer(h, edges3, zeros_n, zeros_n16, W4, a4s, a4d, b4, 1, 32)
    return _make_mlp()(h[0], h[1], Wl1, bl1.reshape(1, 128), Wl2,
                       bl2.reshape(1, 128), Wl3, bl3.reshape(1, 1))


# speedup vs baseline: 1.0406x; 1.0406x over previous
import functools

import jax
import jax.numpy as jnp
from jax import lax
from jax.experimental import pallas as pl
from jax.experimental.pallas import tpu as pltpu
from jax.experimental.pallas import tpu_sc as plsc

N = 100000
E = 3200000
NPAD = 100096
NB = 2000
NW = 32
EPW = E // NW
WIN = 400
NWIN = EPW // WIN
RPT = NPAD // 16

_mesh = lambda: plsc.VectorSubcoreMesh(core_axis_name="c", subcore_axis_name="s")


@functools.cache
def _make_dense(F_in, H, D):
    F = H * D
    nch = F // 16

    def body(x_ref, W_ref, As_ref, Ad_ref, *outs):
        chunk_refs = outs[:nch]
        as_ref = outs[nch]
        ad_ref = outs[nch + 1]
        wself_ref = outs[nch + 2]
        h = jnp.dot(x_ref[...], W_ref[...], preferred_element_type=jnp.float32)
        asv = jnp.dot(h, As_ref[...], preferred_element_type=jnp.float32)
        adv = jnp.dot(h, Ad_ref[...], preferred_element_type=jnp.float32)
        z = asv + adv
        wself_ref[...] = jnp.exp(jnp.maximum(z, 0.2 * z))
        as_ref[...] = asv
        ad_ref[...] = adv
        for k in range(nch):
            chunk_refs[k][...] = h[:, k * 16:(k + 1) * 16]

    grid = (N // NB,)
    in_specs = [
        pl.BlockSpec((NB, F_in), lambda i: (i, 0)),
        pl.BlockSpec((F_in, F), lambda i: (0, 0)),
        pl.BlockSpec((F, H), lambda i: (0, 0)),
        pl.BlockSpec((F, H), lambda i: (0, 0)),
    ]
    out_specs = ([pl.BlockSpec((NB, 16), lambda i: (i, 0))] * nch
                 + [pl.BlockSpec((NB, H), lambda i: (i, 0))] * 3)
    out_shape = ([jax.ShapeDtypeStruct((N, 16), jnp.float32)] * nch
                 + [jax.ShapeDtypeStruct((N, H), jnp.float32)] * 3)
    return pl.pallas_call(body, grid=grid, in_specs=in_specs,
                          out_specs=out_specs, out_shape=out_shape)


@functools.cache
def _make_attn(H):
    per_set = 1 + 3 * H
    scratch = []
    for _ in range(3):
        scratch += [pltpu.VMEM((2, WIN), jnp.int32)]
        scratch += [pltpu.VMEM((WIN,), jnp.float32) for _ in range(3 * H)]
    scratch += [pltpu.VMEM_SHARED((NPAD,), jnp.float32) for _ in range(H)]
    scratch += [pltpu.VMEM((RPT,), jnp.float32)]
    scratch += [pltpu.SemaphoreType.DMA] * 9

    @functools.partial(
        pl.kernel, mesh=_mesh(),
        out_type=[jax.ShapeDtypeStruct((H * E,), jnp.float32),
                  jax.ShapeDtypeStruct((2 * H * NPAD,), jnp.float32)],
        scratch_types=scratch,
        compiler_params=pltpu.CompilerParams(use_tc_tiling_on_sc=False))
    def attn(edges_hbm, *rest):
        atabs = rest[:2 * H]
        zn_hbm = rest[2 * H]
        w_out, den_out = rest[2 * H + 1], rest[2 * H + 2]
        sc = rest[2 * H + 3:]
        sets = []
        for b in range(3):
            o = b * per_set
            sets.append(dict(ebuf=sc[o],
                             asb=sc[o + 1:o + 1 + H],
                             adb=sc[o + 1 + H:o + 1 + 2 * H],
                             wb=sc[o + 1 + 2 * H:o + 1 + 3 * H]))
        dens = sc[3 * per_set:3 * per_set + H]
        vbuf = sc[3 * per_set + H]
        gsem = sc[3 * per_set + H + 1:3 * per_set + H + 4]
        ssem = sc[3 * per_set + H + 4:3 * per_set + H + 7]
        wsem = sc[3 * per_set + H + 7:3 * per_set + H + 10]
        cid = lax.axis_index("c")
        sid = lax.axis_index("s")
        wid = sid * 2 + cid
        pltpu.sync_copy(zn_hbm.at[pl.ds(sid * RPT, RPT)], vbuf)
        for hh in range(H):
            pltpu.sync_copy(vbuf, dens[hh].at[pl.ds(sid * RPT, RPT)])
        plsc.subcore_barrier()

        def issue(j, b):
            s = sets[b]
            pltpu.sync_copy(edges_hbm.at[wid * NWIN + j], s["ebuf"])
            for hh in range(H):
                pltpu.async_copy(atabs[hh].at[s["ebuf"].at[0]], s["asb"][hh],
                                 gsem[b])
                pltpu.async_copy(atabs[H + hh].at[s["ebuf"].at[1]],
                                 s["adb"][hh], gsem[b])

        def drain(b):
            s = sets[b]
            for hh in range(H):
                pltpu.make_async_copy(atabs[hh].at[s["ebuf"].at[0]],
                                      s["asb"][hh], gsem[b]).wait()
                pltpu.make_async_copy(atabs[H + hh].at[s["ebuf"].at[1]],
                                      s["adb"][hh], gsem[b]).wait()

        def retire(b, jprev):
            s = sets[b]
            basep = wid * EPW + jprev * WIN
            for hh in range(H):
                pltpu.make_async_copy(s["wb"][hh],
                                      dens[hh].at[s["ebuf"].at[1]],
                                      ssem[b]).wait()
                pltpu.make_async_copy(s["wb"][hh],
                                      w_out.at[pl.ds(hh * E + basep, WIN)],
                                      wsem[b]).wait()

        def work(j, b):
            s = sets[b]
            base = wid * EPW + j * WIN
            drain(b)
            for hh in range(H):
                def grp(g, c2, hh=hh):
                    z = (s["asb"][hh][pl.ds(g * 16, 16)]
                         + s["adb"][hh][pl.ds(g * 16, 16)])
                    s["wb"][hh][pl.ds(g * 16, 16)] = jnp.exp(
                        jnp.maximum(z, 0.2 * z))
                    return c2

                lax.fori_loop(0, WIN // 16, grp, 0)
                pltpu.async_copy(s["wb"][hh], dens[hh].at[s["ebuf"].at[1]],
                                 ssem[b], add=True)
                pltpu.async_copy(s["wb"][hh],
                                 w_out.at[pl.ds(hh * E + base, WIN)],
                                 wsem[b])

        issue(0, 0)
        issue(1, 1)
        work(0, 0)
        issue(2, 2)
        work(1, 1)

        def triple(k, carry):
            j = 2 + 3 * k
            for t in range(3):
                b = (2 + t) % 3
                retire((b + 1) % 3, j + t - 2)
                issue(j + t + 1, (b + 1) % 3)
                work(j + t, b)
            return carry

        lax.fori_loop(0, (NWIN - 4) // 3, triple, 0)
        retire(0, NWIN - 4)
        issue(NWIN - 1, 0)
        work(NWIN - 2, 2)
        work(NWIN - 1, 0)
        retire(0, NWIN - 1)
        retire(1, NWIN - 3)
        retire(2, NWIN - 2)
        plsc.subcore_barrier()
        for hh in range(H):
            pltpu.sync_copy(dens[hh].at[pl.ds(sid * RPT, RPT)], vbuf)
            pltpu.sync_copy(
                vbuf,
                den_out.at[pl.ds((cid * H + hh) * NPAD + sid * RPT, RPT)])

    return attn


@functools.cache
def _make_agg(H, hh):
    NQ, TAIL = RPT // WIN, RPT % WIN
    scratch = []
    for _ in range(3):
        scratch += [pltpu.VMEM((2, WIN), jnp.int32),
                    pltpu.VMEM((WIN,), jnp.float32),
                    pltpu.VMEM((WIN, 16), jnp.float32)]
    scratch += [pltpu.VMEM_SHARED((NPAD, 16), jnp.float32)]
    scratch += [pltpu.SemaphoreType.DMA] * 6

    @functools.partial(
        pl.kernel, mesh=_mesh(),
        out_type=jax.ShapeDtypeStruct((2, NPAD, 16), jnp.float32),
        scratch_types=scratch,
        compiler_params=pltpu.CompilerParams(use_tc_tiling_on_sc=False))
    def agg(edges_hbm, w_hbm, table_hbm, z16_hbm, out_hbm, *sc):
        sets = [(sc[3 * b], sc[3 * b + 1], sc[3 * b + 2]) for b in range(3)]
        acc = sc[9]
        gsem = sc[10:13]
        ssem = sc[13:16]
        cid = lax.axis_index("c")
        sid = lax.axis_index("s")
        wid = sid * 2 + cid
        rows0 = sets[0][2]
        pltpu.sync_copy(z16_hbm.at[pl.ds(sid * RPT, WIN)], rows0)
        for q in range(NQ):
            pltpu.sync_copy(rows0, acc.at[pl.ds(sid * RPT + q * WIN, WIN)])
        pltpu.sync_copy(rows0.at[pl.ds(0, TAIL)],
                        acc.at[pl.ds(sid * RPT + NQ * WIN, TAIL)])
        plsc.subcore_barrier()

        def issue(j, b):
            eb, wv, rows = sets[b]
            base = wid * EPW + j * WIN
            pltpu.sync_copy(edges_hbm.at[wid * NWIN + j], eb)
            pltpu.async_copy(w_hbm.at[pl.ds(hh * E + base, WIN)], wv,
                             gsem[b])
            pltpu.async_copy(table_hbm.at[eb.at[0]], rows, gsem[b])

        def retire(b):
            eb, wv, rows = sets[b]
            pltpu.make_async_copy(rows, acc.at[eb.at[1]], ssem[b]).wait()

        def work(j, b):
            eb, wv, rows = sets[b]
            base = wid * EPW + j * WIN
            pltpu.make_async_copy(w_hbm.at[pl.ds(hh * E + base, WIN)], wv,
                                  gsem[b]).wait()
            pltpu.make_async_copy(table_hbm.at[eb.at[0]], rows,
                                  gsem[b]).wait()

            def scale(g, c2):
                wv_arr = wv[pl.ds(g * 16, 16)]
                for ei in range(16):
                    r = g * 16 + ei
                    rows[r, :] = rows[r, :] * wv_arr[ei]
                return c2

            lax.fori_loop(0, WIN // 16, scale, 0)
            pltpu.async_copy(rows, acc.at[eb.at[1]], ssem[b], add=True)

        issue(0, 0)
        issue(1, 1)
        work(0, 0)
        issue(2, 2)
        work(1, 1)

        def triple(k, carry):
            j = 2 + 3 * k
            for t in range(3):
                b = (2 + t) % 3
                retire((b + 1) % 3)
                issue(j + t + 1, (b + 1) % 3)
                work(j + t, b)
            return carry

        lax.fori_loop(0, (NWIN - 4) // 3, triple, 0)
        retire(0)
        issue(NWIN - 1, 0)
        work(NWIN - 2, 2)
        work(NWIN - 1, 0)
        for b in range(3):
            retire(b)
        plsc.subcore_barrier()
        for q in range(NQ):
            pltpu.sync_copy(acc.at[pl.ds(sid * RPT + q * WIN, WIN)], rows0)
            pltpu.sync_copy(rows0,
                            out_hbm.at[cid, pl.ds(sid * RPT + q * WIN, WIN)])
        pltpu.sync_copy(acc.at[pl.ds(sid * RPT + NQ * WIN, TAIL)],
                        rows0.at[pl.ds(0, TAIL)])
        pltpu.sync_copy(rows0.at[pl.ds(0, TAIL)],
                        out_hbm.at[cid, pl.ds(sid * RPT + NQ * WIN, TAIL)])

    return agg


@functools.cache
def _make_final(H, D):
    F = H * D
    nch = F // 16

    def body(*refs):
        parts = refs[:nch]
        den_ref = refs[nch]
        wself_ref = refs[nch + 1]
        chunks = refs[nch + 2:2 * nch + 2]
        b_ref = refs[2 * nch + 2]
        out_ref = refs[2 * nch + 3]
        ws = wself_ref[...]
        for k in range(nch):
            hk = (k * 16) // D
            p = parts[k][0] + parts[k][1]
            num = p + ws[:, hk:hk + 1] * chunks[k][...]
            dd = den_ref[0, :, hk] + den_ref[1, :, hk] + ws[:, hk] + 1e-16
            o = num / dd[:, None] + b_ref[0, k * 16:(k + 1) * 16]
            out_ref[:, k * 16:(k + 1) * 16] = jnp.where(
                o > 0, o, jnp.exp(o) - 1.0)

    grid = (N // NB,)
    in_specs = ([pl.BlockSpec((2, NB, 16), lambda i: (0, i, 0))] * nch
                + [pl.BlockSpec((2, NB, H), lambda i: (0, i, 0)),
                   pl.BlockSpec((NB, H), lambda i: (i, 0))]
                + [pl.BlockSpec((NB, 16), lambda i: (i, 0))] * nch
                + [pl.BlockSpec((1, F), lambda i: (0, 0))])
    return pl.pallas_call(
        body, grid=grid, in_specs=in_specs,
        out_specs=pl.BlockSpec((NB, F), lambda i: (i, 0)),
        out_shape=jax.ShapeDtypeStruct((N, F), jnp.float32))


@functools.cache
def _make_mlp():
    def body(x_ref, W1_ref, b1_ref, W2_ref, b2_ref, W3_ref, b3_ref, out_ref):
        h = jnp.maximum(
            jnp.dot(x_ref[...], W1_ref[...],
                    preferred_element_type=jnp.float32) + b1_ref[...], 0.0)
        h = jnp.maximum(
            jnp.dot(h, W2_ref[...],
                    preferred_element_type=jnp.float32) + b2_ref[...], 0.0)
        o = jnp.dot(h, W3_ref[...],
                    preferred_element_type=jnp.float32) + b3_ref[...]
        out_ref[...] = 1.0 / (1.0 + jnp.exp(-o))

    grid = (N // NB,)
    in_specs = [
        pl.BlockSpec((NB, 32), lambda i: (i, 0)),
        pl.BlockSpec((32, 128), lambda i: (0, 0)),
        pl.BlockSpec((1, 128), lambda i: (0, 0)),
        pl.BlockSpec((128, 128), lambda i: (0, 0)),
        pl.BlockSpec((1, 128), lambda i: (0, 0)),
        pl.BlockSpec((128, 1), lambda i: (0, 0)),
        pl.BlockSpec((1, 1), lambda i: (0, 0)),
    ]
    return pl.pallas_call(
        body, grid=grid, in_specs=in_specs,
        out_specs=pl.BlockSpec((NB, 1), lambda i: (i, 0)),
        out_shape=jax.ShapeDtypeStruct((N, 1), jnp.float32))


def _head_expand(a):
    H, D = a.shape
    eye = jnp.eye(H, dtype=a.dtype)
    return (a[:, :, None] * eye[:, None, :]).reshape(H * D, H)


def _gat_layer(h, edges3, zeros_n, zeros_n16, W, a_s, a_d, b, H, D):
    F = H * D
    F_in = h.shape[1]
    outs = _make_dense(F_in, H, D)(h, W, _head_expand(a_s), _head_expand(a_d))
    chunks = outs[:F // 16]
    asv = outs[F // 16]
    adv = outs[F // 16 + 1]
    wself = outs[F // 16 + 2]
    acols = ([asv[:, j] for j in range(H)] + [adv[:, j] for j in range(H)])
    w_all, den = _make_attn(H)(edges3, *acols, zeros_n)
    den = jnp.transpose(den.reshape(2, H, NPAD), (0, 2, 1))
    parts = [_make_agg(H, (k * 16) // D)(edges3, w_all, chunks[k], zeros_n16)
             for k in range(F // 16)]
    return _make_final(H, D)(*parts, den, wself, *chunks, b.reshape(1, F))


def kernel(x, edge_index, W1, a1s, a1d, b1, W2, a2s, a2d, b2, W3, a3s, a3d,
           b3, W4, a4s, a4d, b4, Wl1, bl1, Wl2, bl2, Wl3, bl3):
    er = edge_index.reshape(2, NW, NWIN, WIN)
    edges3 = jnp.transpose(er, (1, 2, 0, 3)).reshape(NW * NWIN, 2, WIN)
    zeros_n = jnp.zeros((NPAD,), jnp.float32)
    zeros_n16 = jnp.zeros((NPAD, 16), jnp.float32)
    h = _gat_layer(x, edges3, zeros_n, zeros_n16, W1, a1s, a1d, b1, 2, 32)
    h = _gat_layer(h, edges3, zeros_n, zeros_n16, W2, a2s, a2d, b2, 2, 64)
    h = _gat_layer(h, edges3, zeros_n, zeros_n16, W3, a3s, a3d, b3, 2, 64)
    h = _gat_layer(h, edges3, zeros_n, zeros_n16, W4, a4s, a4d, b4, 1, 32)
    return _make_mlp()(h, Wl1, bl1.reshape(1, 128), Wl2, bl2.reshape(1, 128),
                       Wl3, bl3.reshape(1, 1))
